# baseline (speedup 1.0000x reference)
import functools

import jax
import jax.numpy as jnp
from jax import lax
from jax.experimental import pallas as pl
from jax.experimental.pallas import tpu as pltpu
from jax.experimental.pallas import tpu_sc as plsc

N = 10000
D = 128
E = 320000

NC = 2
NS = 16
L = 16
NW = NC * NS
FPT = D // NW
NPAD = 10240
NPT = NPAD // NS
CHUNK = 2000
NVEC = CHUNK // L
NCHUNK = E // CHUNK
DEG_E = E // NS
DEG_CH = DEG_E // CHUNK

f32 = jnp.float32
i32 = jnp.int32


def _spl_f(v):
    return jnp.full((L,), v, dtype=f32)


def _spl_i(v):
    return jnp.full((L,), v, dtype=i32)


def _c(v):
    return jnp.int32(v)


def _i(v):
    return v if v.dtype == i32 else lax.convert_element_type(v, i32)


def _fori(n, body, unroll=1):
    def outer(j, carry):
        base = j * _c(unroll)
        for u in range(unroll):
            carry = body(base + _c(u), carry)
        return carry
    lax.fori_loop(_c(0), _c(n // unroll), outer, 0)
    for u in range((n // unroll) * unroll, n):
        body(_c(u), 0)


def _sc_body(row_hbm, col_hbm, xT_hbm, m1T_hbm, m2T_hbm,
             x_sl, agg, deg_priv, dinv, red_buf, erow0, erow1, ecol0, ecol1,
             deg_parts, dinv_s, sem0, sem1, sem_x):
    c = _i(lax.axis_index("c"))
    s = _i(lax.axis_index("s"))
    wid = s * _c(NC) + c
    f0 = wid * _c(FPT)
    sems = (sem0, sem1)
    erows = (erow0, erow1)
    ecols = (ecol0, ecol1)

    xcopy = pltpu.make_async_copy(xT_hbm.at[pl.ds(f0, FPT), :], x_sl, sem_x)
    xcopy.start()

    def zero_deg(j, carry):
        deg_priv[pl.ds(j * _c(L), L)] = _spl_f(0.0)
        return carry

    _fori(NPAD // L, zero_deg, unroll=4)

    def zero_agg(j, carry):
        for k in range(FPT):
            agg[_c(k), pl.ds(j * _c(L), L)] = _spl_f(0.0)
        return carry

    _fori(N // L, zero_agg, unroll=2)

    def edge_sweep(nchunks, base, process_vec):
        def start(chunk_idx, b):
            off = base + chunk_idx * _c(CHUNK)
            pltpu.make_async_copy(row_hbm.at[pl.ds(off, CHUNK)],
                                  erows[b], sems[b]).start()
            pltpu.make_async_copy(col_hbm.at[pl.ds(off, CHUNK)],
                                  ecols[b], sems[b]).start()

        def wait(b):
            pltpu.make_async_copy(row_hbm.at[pl.ds(_c(0), CHUNK)],
                                  erows[b], sems[b]).wait()
            pltpu.make_async_copy(col_hbm.at[pl.ds(_c(0), CHUNK)],
                                  ecols[b], sems[b]).wait()

        def proc_buf(b):
            @plsc.parallel_loop(_c(0), _c(NVEC), _c(1), unroll=8)
            def _(j):
                process_vec(b, j)

        last = _c(nchunks - 1)
        start(_c(0), 0)

        def obody(i, carry):
            g = i * _c(2)
            start(jnp.minimum(g + _c(1), last), 1)
            wait(0)
            proc_buf(0)
            start(jnp.minimum(g + _c(2), last), 0)
            wait(1)
            proc_buf(1)
            return carry

        _fori(nchunks // 2, obody)
        wait(0)

    def deg_vec(b, j):
        sl = pl.ds(j * _c(L), L)
        r = erows[b][sl]
        cl = ecols[b][sl]
        m = jnp.where(r != cl, _spl_f(1.0), _spl_f(0.0))
        plsc.addupdate_scatter(deg_priv, [r], m)

    edge_sweep(DEG_CH, s * _c(DEG_E), deg_vec)

    pltpu.sync_copy(deg_priv, deg_parts.at[s])
    plsc.subcore_barrier()
    nb = s * _c(NPT)
    pltpu.sync_copy(deg_parts.at[:, pl.ds(nb, NPT)], red_buf)

    def red_vec(j, carry):
        sl = pl.ds(j * _c(L), L)
        acc = red_buf[_c(0), sl]
        for t in range(1, NS):
            acc = acc + red_buf[_c(t), sl]
        xi = lax.bitcast_convert_type(acc, i32)
        yi = _spl_i(0x5F3759DF) - lax.shift_right_arithmetic(xi, _spl_i(1))
        y = lax.bitcast_convert_type(yi, f32)
        for _ in range(3):
            y = y * (_spl_f(1.5) - _spl_f(0.5) * acc * y * y)
        y = jnp.where(acc > _spl_f(0.5), y, _spl_f(0.0))
        deg_priv[sl] = y
        return carry

    _fori(NPT // L, red_vec)
    pltpu.sync_copy(deg_priv.at[pl.ds(_c(0), NPT)], dinv_s.at[pl.ds(nb, NPT)])
    plsc.subcore_barrier()
    pltpu.sync_copy(dinv_s, dinv)
    xcopy.wait()

    def scale_vec(j, carry):
        sl = pl.ds(j * _c(L), L)
        d = dinv[sl]
        for k in range(FPT):
            x_sl[_c(k), sl] = x_sl[_c(k), sl] * d
        return carry

    _fori(N // L, scale_vec, unroll=2)

    def pass_vec(b, j):
        sl = pl.ds(j * _c(L), L)
        r = erows[b][sl]
        cl = ecols[b][sl]
        mf = jnp.where(r != cl, _spl_f(1.0), _spl_f(0.0))
        for k in range(FPT):
            kv = _spl_i(k)
            xv = plsc.load_gather(x_sl, [kv, cl])
            plsc.addupdate_scatter(agg, [kv, r], mf * xv)

    def fin_vec(j, carry):
        sl = pl.ds(j * _c(L), L)
        d = dinv[sl]
        for k in range(FPT):
            x_sl[_c(k), sl] = agg[_c(k), sl] * d
            agg[_c(k), sl] = _spl_f(0.0)
        return carry

    def rescale_vec(j, carry):
        sl = pl.ds(j * _c(L), L)
        d = dinv[sl]
        for k in range(FPT):
            x_sl[_c(k), sl] = x_sl[_c(k), sl] * d
        return carry

    edge_sweep(NCHUNK, 0, pass_vec)
    _fori(N // L, fin_vec, unroll=2)
    pltpu.sync_copy(x_sl, m1T_hbm.at[pl.ds(f0, FPT), :])
    _fori(N // L, rescale_vec, unroll=2)

    edge_sweep(NCHUNK, 0, pass_vec)
    _fori(N // L, fin_vec, unroll=2)
    pltpu.sync_copy(x_sl, m2T_hbm.at[pl.ds(f0, FPT), :])


_sc_spmm = pl.kernel(
    _sc_body,
    out_type=(jax.ShapeDtypeStruct((D, N), f32),
              jax.ShapeDtypeStruct((D, N), f32)),
    mesh=plsc.VectorSubcoreMesh(core_axis_name="c", subcore_axis_name="s",
                                num_cores=NC, num_subcores=NS),
    compiler_params=pltpu.CompilerParams(needs_layout_passes=False),
    scratch_types=[
        pltpu.VMEM((FPT, N), f32),
        pltpu.VMEM((FPT, N), f32),
        pltpu.VMEM((NPAD,), f32),
        pltpu.VMEM((NPAD,), f32),
        pltpu.VMEM((NS, NPT), f32),
        pltpu.VMEM((CHUNK,), i32),
        pltpu.VMEM((CHUNK,), i32),
        pltpu.VMEM((CHUNK,), i32),
        pltpu.VMEM((CHUNK,), i32),
        pltpu.VMEM_SHARED((NS, NPAD), f32),
        pltpu.VMEM_SHARED((NPAD,), f32),
        pltpu.SemaphoreType.DMA,
        pltpu.SemaphoreType.DMA,
        pltpu.SemaphoreType.DMA,
    ],
)


BN = 1000


def _tc_body(cs_ref, xT_ref, m1T_ref, m2T_ref, w_ref, bias_ref, out_ref):
    c0 = cs_ref[0]
    c1 = cs_ref[1]
    c2 = cs_ref[2]
    comb = c0 * xT_ref[...] + c1 * m1T_ref[...] + c2 * m2T_ref[...]
    out_ref[...] = lax.dot_general(
        comb, w_ref[...], (((0,), (1,)), ((), ())),
        preferred_element_type=f32) + bias_ref[...]


_tc_combine = pl.pallas_call(
    _tc_body,
    in_specs=[
        pl.BlockSpec(memory_space=pltpu.SMEM),
        pl.BlockSpec((D, N), lambda: (0, 0)),
        pl.BlockSpec((D, N), lambda: (0, 0)),
        pl.BlockSpec((D, N), lambda: (0, 0)),
        pl.BlockSpec((D, D), lambda: (0, 0)),
        pl.BlockSpec((1, D), lambda: (0, 0)),
    ],
    out_specs=pl.BlockSpec((N, D), lambda: (0, 0)),
    out_shape=jax.ShapeDtypeStruct((N, D), f32),
)


def kernel(x, edge_index, W, bias, a, b, c):
    row = edge_index[0].astype(i32)
    col = edge_index[1].astype(i32)
    xT = x.T
    m1T, m2T = _sc_spmm(row, col, xT)
    cs = jnp.stack([a + b + c, -(2.0 * a + b), a]).astype(f32)
    return _tc_combine(cs, xT, m1T, m2T, W, bias.reshape(1, D))


# speedup vs baseline: 1.0737x; 1.0737x over previous
import functools

import jax
import jax.numpy as jnp
from jax import lax
from jax.experimental import pallas as pl
from jax.experimental.pallas import tpu as pltpu
from jax.experimental.pallas import tpu_sc as plsc

N = 10000
D = 128
E = 320000

NC = 2
NS = 16
L = 16
NW = NC * NS
FPT = D // NW
NPAD = 10240
NPT = NPAD // NS
CHUNK = 2000
NVEC = CHUNK // L
NCHUNK = E // CHUNK
DEG_E = E // NS
DEG_CH = DEG_E // CHUNK

f32 = jnp.float32
i32 = jnp.int32


def _spl_f(v):
    return jnp.full((L,), v, dtype=f32)


def _spl_i(v):
    return jnp.full((L,), v, dtype=i32)


def _c(v):
    return jnp.int32(v)


def _i(v):
    return v if v.dtype == i32 else lax.convert_element_type(v, i32)


def _fori(n, body, unroll=1):
    def outer(j, carry):
        base = j * _c(unroll)
        for u in range(unroll):
            carry = body(base + _c(u), carry)
        return carry
    lax.fori_loop(_c(0), _c(n // unroll), outer, 0)
    for u in range((n // unroll) * unroll, n):
        body(_c(u), 0)


def _sc_body(row_hbm, col_hbm, xT_hbm, m1T_hbm, m2T_hbm,
             x_sl, agg, deg_priv, dinv, red_buf, erow0, erow1, ecol0, ecol1,
             deg_parts, dinv_s, sem0, sem1, sem_x):
    c = _i(lax.axis_index("c"))
    s = _i(lax.axis_index("s"))
    wid = s * _c(NC) + c
    f0 = wid * _c(FPT)
    sems = (sem0, sem1)
    erows = (erow0, erow1)
    ecols = (ecol0, ecol1)

    xcopy = pltpu.make_async_copy(xT_hbm.at[pl.ds(f0, FPT), :], x_sl, sem_x)
    xcopy.start()

    def zero_deg(j, carry):
        deg_priv[pl.ds(j * _c(L), L)] = _spl_f(0.0)
        return carry

    _fori(NPAD // L, zero_deg, unroll=4)

    def zero_agg(j, carry):
        for k in range(FPT):
            agg[_c(k), pl.ds(j * _c(L), L)] = _spl_f(0.0)
        return carry

    _fori(N // L, zero_agg, unroll=2)

    def edge_sweep(nchunks, base, process_vec):
        def start(chunk_idx, b):
            off = base + chunk_idx * _c(CHUNK)
            pltpu.make_async_copy(row_hbm.at[pl.ds(off, CHUNK)],
                                  erows[b], sems[b]).start()
            pltpu.make_async_copy(col_hbm.at[pl.ds(off, CHUNK)],
                                  ecols[b], sems[b]).start()

        def wait(b):
            pltpu.make_async_copy(row_hbm.at[pl.ds(_c(0), CHUNK)],
                                  erows[b], sems[b]).wait()
            pltpu.make_async_copy(col_hbm.at[pl.ds(_c(0), CHUNK)],
                                  ecols[b], sems[b]).wait()

        def proc_buf(b):
            @plsc.parallel_loop(_c(0), _c(NVEC), _c(1), unroll=4)
            def _(j):
                process_vec(b, j)

        last = _c(nchunks - 1)
        start(_c(0), 0)

        def obody(i, carry):
            g = i * _c(2)
            start(jnp.minimum(g + _c(1), last), 1)
            wait(0)
            proc_buf(0)
            start(jnp.minimum(g + _c(2), last), 0)
            wait(1)
            proc_buf(1)
            return carry

        _fori(nchunks // 2, obody)
        wait(0)

    def deg_vec(b, j):
        sl = pl.ds(j * _c(L), L)
        r = erows[b][sl]
        cl = ecols[b][sl]
        m = jnp.where(r != cl, _spl_f(1.0), _spl_f(0.0))
        plsc.addupdate_scatter(deg_priv, [r], m)

    edge_sweep(DEG_CH, s * _c(DEG_E), deg_vec)

    pltpu.sync_copy(deg_priv, deg_parts.at[s])
    plsc.subcore_barrier()
    nb = s * _c(NPT)
    pltpu.sync_copy(deg_parts.at[:, pl.ds(nb, NPT)], red_buf)

    def red_vec(j, carry):
        sl = pl.ds(j * _c(L), L)
        acc = red_buf[_c(0), sl]
        for t in range(1, NS):
            acc = acc + red_buf[_c(t), sl]
        xi = lax.bitcast_convert_type(acc, i32)
        yi = _spl_i(0x5F3759DF) - lax.shift_right_arithmetic(xi, _spl_i(1))
        y = lax.bitcast_convert_type(yi, f32)
        for _ in range(3):
            y = y * (_spl_f(1.5) - _spl_f(0.5) * acc * y * y)
        y = jnp.where(acc > _spl_f(0.5), y, _spl_f(0.0))
        deg_priv[sl] = y
        return carry

    _fori(NPT // L, red_vec)
    pltpu.sync_copy(deg_priv.at[pl.ds(_c(0), NPT)], dinv_s.at[pl.ds(nb, NPT)])
    plsc.subcore_barrier()
    pltpu.sync_copy(dinv_s, dinv)
    xcopy.wait()

    def scale_vec(j, carry):
        sl = pl.ds(j * _c(L), L)
        d = dinv[sl]
        for k in range(FPT):
            x_sl[_c(k), sl] = x_sl[_c(k), sl] * d
        return carry

    _fori(N // L, scale_vec, unroll=2)

    def pass_vec(b, j):
        sl = pl.ds(j * _c(L), L)
        r = erows[b][sl]
        cl = ecols[b][sl]
        mf = jnp.where(r != cl, _spl_f(1.0), _spl_f(0.0))
        for k in range(FPT):
            kv = _spl_i(k)
            xv = plsc.load_gather(x_sl, [kv, cl])
            plsc.addupdate_scatter(agg, [kv, r], mf * xv)

    def fin_vec(j, carry):
        sl = pl.ds(j * _c(L), L)
        d = dinv[sl]
        for k in range(FPT):
            x_sl[_c(k), sl] = agg[_c(k), sl] * d
            agg[_c(k), sl] = _spl_f(0.0)
        return carry

    def rescale_vec(j, carry):
        sl = pl.ds(j * _c(L), L)
        d = dinv[sl]
        for k in range(FPT):
            x_sl[_c(k), sl] = x_sl[_c(k), sl] * d
        return carry

    edge_sweep(NCHUNK, 0, pass_vec)
    _fori(N // L, fin_vec, unroll=2)
    pltpu.sync_copy(x_sl, m1T_hbm.at[pl.ds(f0, FPT), :])
    _fori(N // L, rescale_vec, unroll=2)

    edge_sweep(NCHUNK, 0, pass_vec)
    _fori(N // L, fin_vec, unroll=2)
    pltpu.sync_copy(x_sl, m2T_hbm.at[pl.ds(f0, FPT), :])


_sc_spmm = pl.kernel(
    _sc_body,
    out_type=(jax.ShapeDtypeStruct((D, N), f32),
              jax.ShapeDtypeStruct((D, N), f32)),
    mesh=plsc.VectorSubcoreMesh(core_axis_name="c", subcore_axis_name="s",
                                num_cores=NC, num_subcores=NS),
    compiler_params=pltpu.CompilerParams(needs_layout_passes=False),
    scratch_types=[
        pltpu.VMEM((FPT, N), f32),
        pltpu.VMEM((FPT, N), f32),
        pltpu.VMEM((NPAD,), f32),
        pltpu.VMEM((NPAD,), f32),
        pltpu.VMEM((NS, NPT), f32),
        pltpu.VMEM((CHUNK,), i32),
        pltpu.VMEM((CHUNK,), i32),
        pltpu.VMEM((CHUNK,), i32),
        pltpu.VMEM((CHUNK,), i32),
        pltpu.VMEM_SHARED((NS, NPAD), f32),
        pltpu.VMEM_SHARED((NPAD,), f32),
        pltpu.SemaphoreType.DMA,
        pltpu.SemaphoreType.DMA,
        pltpu.SemaphoreType.DMA,
    ],
)


BN = 1000


def _tc_base_body(c0_ref, x_ref, w_ref, bias_ref, out_ref):
    out_ref[...] = c0_ref[0] * lax.dot_general(
        x_ref[...], w_ref[...], (((1,), (1,)), ((), ())),
        preferred_element_type=f32) + bias_ref[...]


_tc_base = pl.pallas_call(
    _tc_base_body,
    in_specs=[
        pl.BlockSpec(memory_space=pltpu.SMEM),
        pl.BlockSpec((N, D), lambda: (0, 0)),
        pl.BlockSpec((D, D), lambda: (0, 0)),
        pl.BlockSpec((1, D), lambda: (0, 0)),
    ],
    out_specs=pl.BlockSpec((N, D), lambda: (0, 0)),
    out_shape=jax.ShapeDtypeStruct((N, D), f32),
)


def _tc_add_body(cs_ref, m1T_ref, m2T_ref, w_ref, base_ref, out_ref):
    comb = cs_ref[0] * m1T_ref[...] + cs_ref[1] * m2T_ref[...]
    out_ref[...] = base_ref[...] + lax.dot_general(
        comb, w_ref[...], (((0,), (1,)), ((), ())),
        preferred_element_type=f32)


_tc_add = pl.pallas_call(
    _tc_add_body,
    in_specs=[
        pl.BlockSpec(memory_space=pltpu.SMEM),
        pl.BlockSpec((D, N), lambda: (0, 0)),
        pl.BlockSpec((D, N), lambda: (0, 0)),
        pl.BlockSpec((D, D), lambda: (0, 0)),
        pl.BlockSpec((N, D), lambda: (0, 0)),
    ],
    out_specs=pl.BlockSpec((N, D), lambda: (0, 0)),
    out_shape=jax.ShapeDtypeStruct((N, D), f32),
)


def kernel(x, edge_index, W, bias, a, b, c):
    row = edge_index[0].astype(i32)
    col = edge_index[1].astype(i32)
    xT = x.T
    m1T, m2T = _sc_spmm(row, col, xT)
    c0 = jnp.reshape(a + b + c, (1,)).astype(f32)
    cs = jnp.stack([-(2.0 * a + b), a]).astype(f32)
    base = _tc_base(c0, x, W, bias.reshape(1, D))
    return _tc_add(cs, m1T, m2T, W, base)


# speedup vs baseline: 1.1812x; 1.1002x over previous
import functools

import jax
import jax.numpy as jnp
from jax import lax
from jax.experimental import pallas as pl
from jax.experimental.pallas import tpu as pltpu
from jax.experimental.pallas import tpu_sc as plsc

N = 10000
D = 128
E = 320000

NC = 2
NS = 16
L = 16
NW = NC * NS
FPT = D // NW
NSTRIDE = 10008
NPAD = 10240
NPT = NPAD // NS
CHUNK = 2000
NVEC = CHUNK // L
NCHUNK = E // CHUNK
DEG_E = E // NS
DEG_CH = DEG_E // CHUNK

f32 = jnp.float32
i32 = jnp.int32


def _spl_f(v):
    return jnp.full((L,), v, dtype=f32)


def _spl_i(v):
    return jnp.full((L,), v, dtype=i32)


def _c(v):
    return jnp.int32(v)


def _i(v):
    return v if v.dtype == i32 else lax.convert_element_type(v, i32)


_PERM_DNUMS = lax.GatherDimensionNumbers(
    offset_dims=(), collapsed_slice_dims=(0,), start_index_map=(0,))


def _lperm(v, idx):
    return lax.gather(v, idx.reshape(L, 1), _PERM_DNUMS, (1,),
                      mode=lax.GatherScatterMode.PROMISE_IN_BOUNDS)


def _fori(n, body, unroll=1):
    def outer(j, carry):
        base = j * _c(unroll)
        for u in range(unroll):
            carry = body(base + _c(u), carry)
        return carry
    lax.fori_loop(_c(0), _c(n // unroll), outer, 0)
    for u in range((n // unroll) * unroll, n):
        body(_c(u), 0)


def _sc_body(row_hbm, col_hbm, xT_hbm, m1T_hbm, m2T_hbm,
             x_sl, agg, deg_priv, dinv, red_buf, erow0, erow1, ecol0, ecol1,
             deg_parts, dinv_s, sem0, sem1, sem_x):
    c = _i(lax.axis_index("c"))
    s = _i(lax.axis_index("s"))
    wid = s * _c(NC) + c
    f0 = wid * _c(FPT)
    sems = (sem0, sem1)
    erows = (erow0, erow1)
    ecols = (ecol0, ecol1)

    xcopies = [
        pltpu.make_async_copy(xT_hbm.at[pl.ds((f0 + _c(k)) * _c(N), N)],
                              x_sl.at[pl.ds(_c(k * NSTRIDE), N)], sem_x)
        for k in range(FPT)
    ]
    for cp in xcopies:
        cp.start()

    def zero_deg(j, carry):
        deg_priv[pl.ds(j * _c(L), L)] = _spl_f(0.0)
        return carry

    _fori(NPAD // L, zero_deg, unroll=4)

    def zero_agg(j, carry):
        agg[pl.ds(j * _c(L), L)] = _spl_f(0.0)
        return carry

    _fori(FPT * NSTRIDE // L, zero_agg, unroll=4)

    j16 = lax.iota(i32, 16)
    e8 = lax.shift_right_logical(j16, _spl_i(1))
    phs = (e8, e8 + _spl_i(8))
    kp01 = (j16 & _spl_i(1)) * _spl_i(NSTRIDE)
    kps = (kp01, kp01 + _spl_i(2 * NSTRIDE))

    def edge_sweep(nchunks, base, process_vec):
        def start(chunk_idx, b):
            off = base + chunk_idx * _c(CHUNK)
            pltpu.make_async_copy(row_hbm.at[pl.ds(off, CHUNK)],
                                  erows[b], sems[b]).start()
            pltpu.make_async_copy(col_hbm.at[pl.ds(off, CHUNK)],
                                  ecols[b], sems[b]).start()

        def wait(b):
            pltpu.make_async_copy(row_hbm.at[pl.ds(_c(0), CHUNK)],
                                  erows[b], sems[b]).wait()
            pltpu.make_async_copy(col_hbm.at[pl.ds(_c(0), CHUNK)],
                                  ecols[b], sems[b]).wait()

        def proc_buf(b):
            @plsc.parallel_loop(_c(0), _c(NVEC), _c(1), unroll=4)
            def _(j):
                process_vec(b, j)

        last = _c(nchunks - 1)
        start(_c(0), 0)

        def obody(i, carry):
            g = i * _c(2)
            start(jnp.minimum(g + _c(1), last), 1)
            wait(0)
            proc_buf(0)
            start(jnp.minimum(g + _c(2), last), 0)
            wait(1)
            proc_buf(1)
            return carry

        _fori(nchunks // 2, obody)
        wait(0)

    def deg_vec(b, j):
        sl = pl.ds(j * _c(L), L)
        r = erows[b][sl]
        cl = ecols[b][sl]
        m = jnp.where(r != cl, _spl_f(1.0), _spl_f(0.0))
        plsc.addupdate_scatter(deg_priv, [r], m)

    edge_sweep(DEG_CH, s * _c(DEG_E), deg_vec)

    pltpu.sync_copy(deg_priv, deg_parts.at[s])
    plsc.subcore_barrier()
    nb = s * _c(NPT)
    pltpu.sync_copy(deg_parts.at[:, pl.ds(nb, NPT)], red_buf)

    def red_vec(j, carry):
        sl = pl.ds(j * _c(L), L)
        acc = red_buf[_c(0), sl]
        for t in range(1, NS):
            acc = acc + red_buf[_c(t), sl]
        xi = lax.bitcast_convert_type(acc, i32)
        yi = _spl_i(0x5F3759DF) - lax.shift_right_arithmetic(xi, _spl_i(1))
        y = lax.bitcast_convert_type(yi, f32)
        for _ in range(3):
            y = y * (_spl_f(1.5) - _spl_f(0.5) * acc * y * y)
        y = jnp.where(acc > _spl_f(0.5), y, _spl_f(0.0))
        deg_priv[sl] = y
        return carry

    _fori(NPT // L, red_vec)
    pltpu.sync_copy(deg_priv.at[pl.ds(_c(0), NPT)], dinv_s.at[pl.ds(nb, NPT)])
    plsc.subcore_barrier()
    pltpu.sync_copy(dinv_s, dinv)
    for cp in xcopies:
        cp.wait()

    def scale_vec(j, carry):
        sl = pl.ds(j * _c(L), L)
        d = dinv[sl]
        for k in range(FPT):
            off = pl.ds(j * _c(L) + _c(k * NSTRIDE), L)
            x_sl[off] = x_sl[off] * d
        return carry

    _fori(N // L, scale_vec, unroll=2)

    def pass_vec(b, j):
        sl = pl.ds(j * _c(L), L)
        r = erows[b][sl]
        cl = ecols[b][sl]
        for ph in phs:
            rg = _lperm(r, ph)
            cg = _lperm(cl, ph)
            mf = jnp.where(rg != cg, _spl_f(1.0), _spl_f(0.0))
            for kp in kps:
                xv = plsc.load_gather(x_sl, [cg + kp])
                plsc.addupdate_scatter(agg, [rg + kp], mf * xv)

    def fin_vec(j, carry):
        sl = pl.ds(j * _c(L), L)
        d = dinv[sl]
        for k in range(FPT):
            off = pl.ds(j * _c(L) + _c(k * NSTRIDE), L)
            x_sl[off] = agg[off] * d
            agg[off] = _spl_f(0.0)
        return carry

    def rescale_vec(j, carry):
        sl = pl.ds(j * _c(L), L)
        d = dinv[sl]
        for k in range(FPT):
            off = pl.ds(j * _c(L) + _c(k * NSTRIDE), L)
            x_sl[off] = x_sl[off] * d
        return carry

    edge_sweep(NCHUNK, 0, pass_vec)
    _fori(N // L, fin_vec, unroll=2)
    for k in range(FPT):
        pltpu.sync_copy(x_sl.at[pl.ds(_c(k * NSTRIDE), N)],
                        m1T_hbm.at[pl.ds((f0 + _c(k)) * _c(N), N)])
    _fori(N // L, rescale_vec, unroll=2)

    edge_sweep(NCHUNK, 0, pass_vec)
    _fori(N // L, fin_vec, unroll=2)
    for k in range(FPT):
        pltpu.sync_copy(x_sl.at[pl.ds(_c(k * NSTRIDE), N)],
                        m2T_hbm.at[pl.ds((f0 + _c(k)) * _c(N), N)])


_sc_spmm = pl.kernel(
    _sc_body,
    out_type=(jax.ShapeDtypeStruct((D * N,), f32),
              jax.ShapeDtypeStruct((D * N,), f32)),
    mesh=plsc.VectorSubcoreMesh(core_axis_name="c", subcore_axis_name="s",
                                num_cores=NC, num_subcores=NS),
    compiler_params=pltpu.CompilerParams(needs_layout_passes=False),
    scratch_types=[
        pltpu.VMEM((FPT * NSTRIDE,), f32),
        pltpu.VMEM((FPT * NSTRIDE,), f32),
        pltpu.VMEM((NPAD,), f32),
        pltpu.VMEM((NPAD,), f32),
        pltpu.VMEM((NS, NPT), f32),
        pltpu.VMEM((CHUNK,), i32),
        pltpu.VMEM((CHUNK,), i32),
        pltpu.VMEM((CHUNK,), i32),
        pltpu.VMEM((CHUNK,), i32),
        pltpu.VMEM_SHARED((NS, NPAD), f32),
        pltpu.VMEM_SHARED((NPAD,), f32),
        pltpu.SemaphoreType.DMA,
        pltpu.SemaphoreType.DMA,
        pltpu.SemaphoreType.DMA,
    ],
)


BN = 1000


def _tc_body(cs_ref, xT_ref, m1T_ref, m2T_ref, w_ref, bias_ref, out_ref):
    c0 = cs_ref[0]
    c1 = cs_ref[1]
    c2 = cs_ref[2]
    comb = c0 * xT_ref[...] + c1 * m1T_ref[...] + c2 * m2T_ref[...]
    out_ref[...] = lax.dot_general(
        comb, w_ref[...], (((0,), (1,)), ((), ())),
        preferred_element_type=f32) + bias_ref[...]


_tc_combine = pl.pallas_call(
    _tc_body,
    in_specs=[
        pl.BlockSpec(memory_space=pltpu.SMEM),
        pl.BlockSpec((D, N), lambda: (0, 0)),
        pl.BlockSpec((D, N), lambda: (0, 0)),
        pl.BlockSpec((D, N), lambda: (0, 0)),
        pl.BlockSpec((D, D), lambda: (0, 0)),
        pl.BlockSpec((1, D), lambda: (0, 0)),
    ],
    out_specs=pl.BlockSpec((N, D), lambda: (0, 0)),
    out_shape=jax.ShapeDtypeStruct((N, D), f32),
)


def kernel(x, edge_index, W, bias, a, b, c):
    row = edge_index[0].astype(i32)
    col = edge_index[1].astype(i32)
    xT = x.T
    m1f, m2f = _sc_spmm(row, col, xT.reshape(D * N))
    cs = jnp.stack([a + b + c, -(2.0 * a + b), a]).astype(f32)
    return _tc_combine(cs, xT, m1f.reshape(D, N), m2f.reshape(D, N),
                       W, bias.reshape(1, D))


# speedup vs baseline: 1.2168x; 1.0301x over previous
import functools

import jax
import jax.numpy as jnp
from jax import lax
from jax.experimental import pallas as pl
from jax.experimental.pallas import tpu as pltpu
from jax.experimental.pallas import tpu_sc as plsc

N = 10000
D = 128
E = 320000

NC = 2
NS = 16
L = 16
NW = NC * NS
FPT = D // NW
NSTRIDE = 10008
NPAD = 10240
NPT = NPAD // NS
CHUNK = 2000
NVEC = CHUNK // L
NCHUNK = E // CHUNK
DEG_E = E // NS
DEG_CH = DEG_E // CHUNK

f32 = jnp.float32
i32 = jnp.int32


def _spl_f(v):
    return jnp.full((L,), v, dtype=f32)


def _spl_i(v):
    return jnp.full((L,), v, dtype=i32)


def _c(v):
    return jnp.int32(v)


def _i(v):
    return v if v.dtype == i32 else lax.convert_element_type(v, i32)


_PERM_DNUMS = lax.GatherDimensionNumbers(
    offset_dims=(), collapsed_slice_dims=(0,), start_index_map=(0,))


def _lperm(v, idx):
    return lax.gather(v, idx.reshape(L, 1), _PERM_DNUMS, (1,),
                      mode=lax.GatherScatterMode.PROMISE_IN_BOUNDS)


def _fori(n, body, unroll=1):
    def outer(j, carry):
        base = j * _c(unroll)
        for u in range(unroll):
            carry = body(base + _c(u), carry)
        return carry
    lax.fori_loop(_c(0), _c(n // unroll), outer, 0)
    for u in range((n // unroll) * unroll, n):
        body(_c(u), 0)


def _sc_body(ep_hbm, xT_hbm, m1T_hbm, m2T_hbm,
             x_sl, agg, deg_priv, dinv, red_buf, ep0, ep1,
             deg_parts, dinv_s, sem0, sem1, sem_x):
    c = _i(lax.axis_index("c"))
    s = _i(lax.axis_index("s"))
    wid = s * _c(NC) + c
    f0 = wid * _c(FPT)
    sems = (sem0, sem1)
    ebufs = (ep0, ep1)
    c14 = _spl_i(14)
    cmask = _spl_i((1 << 14) - 1)

    xcopies = [
        pltpu.make_async_copy(xT_hbm.at[pl.ds((f0 + _c(k)) * _c(N), N)],
                              x_sl.at[pl.ds(_c(k * NSTRIDE), N)], sem_x)
        for k in range(FPT)
    ]
    for cp in xcopies:
        cp.start()

    def zero_deg(j, carry):
        deg_priv[pl.ds(j * _c(L), L)] = _spl_f(0.0)
        return carry

    _fori(NPAD // L, zero_deg, unroll=4)

    def zero_agg(j, carry):
        agg[pl.ds(j * _c(L), L)] = _spl_f(0.0)
        return carry

    _fori(FPT * NSTRIDE // L, zero_agg, unroll=4)

    j16 = lax.iota(i32, 16)
    e8 = lax.shift_right_logical(j16, _spl_i(1))
    phs = (e8, e8 + _spl_i(8))
    kp01 = (j16 & _spl_i(1)) * _spl_i(NSTRIDE)
    kps = (kp01, kp01 + _spl_i(2 * NSTRIDE))

    def edge_sweep(nchunks, base, process_vec):
        def start(chunk_idx, b):
            off = base + chunk_idx * _c(CHUNK)
            pltpu.make_async_copy(ep_hbm.at[pl.ds(off, CHUNK)],
                                  ebufs[b], sems[b]).start()

        def wait(b):
            pltpu.make_async_copy(ep_hbm.at[pl.ds(_c(0), CHUNK)],
                                  ebufs[b], sems[b]).wait()

        def proc_buf(b):
            @plsc.parallel_loop(_c(0), _c(NVEC), _c(1), unroll=4)
            def _(j):
                process_vec(b, j)

        last = _c(nchunks - 1)
        start(_c(0), 0)

        def obody(i, carry):
            g = i * _c(2)
            start(jnp.minimum(g + _c(1), last), 1)
            wait(0)
            proc_buf(0)
            start(jnp.minimum(g + _c(2), last), 0)
            wait(1)
            proc_buf(1)
            return carry

        _fori(nchunks // 2, obody)
        wait(0)

    def deg_vec(b, j):
        sl = pl.ds(j * _c(L), L)
        ep = ebufs[b][sl]
        r = lax.shift_right_logical(ep, c14)
        cl = ep & cmask
        m = jnp.where(r != cl, _spl_f(1.0), _spl_f(0.0))
        plsc.addupdate_scatter(deg_priv, [r], m)

    edge_sweep(DEG_CH, s * _c(DEG_E), deg_vec)

    pltpu.sync_copy(deg_priv, deg_parts.at[s])
    plsc.subcore_barrier()
    nb = s * _c(NPT)
    pltpu.sync_copy(deg_parts.at[:, pl.ds(nb, NPT)], red_buf)

    def red_vec(j, carry):
        sl = pl.ds(j * _c(L), L)
        acc = red_buf[_c(0), sl]
        for t in range(1, NS):
            acc = acc + red_buf[_c(t), sl]
        xi = lax.bitcast_convert_type(acc, i32)
        yi = _spl_i(0x5F3759DF) - lax.shift_right_arithmetic(xi, _spl_i(1))
        y = lax.bitcast_convert_type(yi, f32)
        for _ in range(3):
            y = y * (_spl_f(1.5) - _spl_f(0.5) * acc * y * y)
        y = jnp.where(acc > _spl_f(0.5), y, _spl_f(0.0))
        deg_priv[sl] = y
        return carry

    _fori(NPT // L, red_vec)
    pltpu.sync_copy(deg_priv.at[pl.ds(_c(0), NPT)], dinv_s.at[pl.ds(nb, NPT)])
    plsc.subcore_barrier()
    pltpu.sync_copy(dinv_s, dinv)
    for cp in xcopies:
        cp.wait()

    def scale_vec(j, carry):
        sl = pl.ds(j * _c(L), L)
        d = dinv[sl]
        for k in range(FPT):
            off = pl.ds(j * _c(L) + _c(k * NSTRIDE), L)
            x_sl[off] = x_sl[off] * d
        return carry

    _fori(N // L, scale_vec, unroll=2)

    def pass_vec(b, j):
        sl = pl.ds(j * _c(L), L)
        ep = ebufs[b][sl]
        for ph in phs:
            eg = _lperm(ep, ph)
            rg = lax.shift_right_logical(eg, c14)
            cg = eg & cmask
            mf = jnp.where(rg != cg, _spl_f(1.0), _spl_f(0.0))
            for kp in kps:
                xv = plsc.load_gather(x_sl, [cg + kp])
                plsc.addupdate_scatter(agg, [rg + kp], mf * xv)

    def fin_vec(j, carry):
        sl = pl.ds(j * _c(L), L)
        d = dinv[sl]
        for k in range(FPT):
            off = pl.ds(j * _c(L) + _c(k * NSTRIDE), L)
            x_sl[off] = agg[off] * d
            agg[off] = _spl_f(0.0)
        return carry

    def rescale_vec(j, carry):
        sl = pl.ds(j * _c(L), L)
        d = dinv[sl]
        for k in range(FPT):
            off = pl.ds(j * _c(L) + _c(k * NSTRIDE), L)
            x_sl[off] = x_sl[off] * d
        return carry

    edge_sweep(NCHUNK, 0, pass_vec)
    _fori(N // L, fin_vec, unroll=2)
    for k in range(FPT):
        pltpu.sync_copy(x_sl.at[pl.ds(_c(k * NSTRIDE), N)],
                        m1T_hbm.at[pl.ds((f0 + _c(k)) * _c(N), N)])
    _fori(N // L, rescale_vec, unroll=2)

    edge_sweep(NCHUNK, 0, pass_vec)
    _fori(N // L, fin_vec, unroll=2)
    for k in range(FPT):
        pltpu.sync_copy(x_sl.at[pl.ds(_c(k * NSTRIDE), N)],
                        m2T_hbm.at[pl.ds((f0 + _c(k)) * _c(N), N)])


_sc_spmm = pl.kernel(
    _sc_body,
    out_type=(jax.ShapeDtypeStruct((D * N,), f32),
              jax.ShapeDtypeStruct((D * N,), f32)),
    mesh=plsc.VectorSubcoreMesh(core_axis_name="c", subcore_axis_name="s",
                                num_cores=NC, num_subcores=NS),
    compiler_params=pltpu.CompilerParams(needs_layout_passes=False),
    scratch_types=[
        pltpu.VMEM((FPT * NSTRIDE,), f32),
        pltpu.VMEM((FPT * NSTRIDE,), f32),
        pltpu.VMEM((NPAD,), f32),
        pltpu.VMEM((NPAD,), f32),
        pltpu.VMEM((NS, NPT), f32),
        pltpu.VMEM((CHUNK,), i32),
        pltpu.VMEM((CHUNK,), i32),
        pltpu.VMEM_SHARED((NS, NPAD), f32),
        pltpu.VMEM_SHARED((NPAD,), f32),
        pltpu.SemaphoreType.DMA,
        pltpu.SemaphoreType.DMA,
        pltpu.SemaphoreType.DMA,
    ],
)


BN = 1000


EPR = E // 128


def _pack_body(r_ref, c_ref, out_ref):
    out_ref[...] = jnp.bitwise_or(
        lax.shift_left(r_ref[...], jnp.int32(14)), c_ref[...])


_tc_pack = pl.pallas_call(
    _pack_body,
    in_specs=[pl.BlockSpec((EPR, 128), lambda: (0, 0)),
              pl.BlockSpec((EPR, 128), lambda: (0, 0))],
    out_specs=pl.BlockSpec((EPR, 128), lambda: (0, 0)),
    out_shape=jax.ShapeDtypeStruct((EPR, 128), i32),
)


def _tc_body(cs_ref, xT_ref, m1T_ref, m2T_ref, w_ref, bias_ref, out_ref):
    c0 = cs_ref[0]
    c1 = cs_ref[1]
    c2 = cs_ref[2]
    comb = c0 * xT_ref[...] + c1 * m1T_ref[...] + c2 * m2T_ref[...]
    out_ref[...] = lax.dot_general(
        comb, w_ref[...], (((0,), (1,)), ((), ())),
        preferred_element_type=f32) + bias_ref[...]


_tc_combine = pl.pallas_call(
    _tc_body,
    in_specs=[
        pl.BlockSpec(memory_space=pltpu.SMEM),
        pl.BlockSpec((D, N), lambda: (0, 0)),
        pl.BlockSpec((D, N), lambda: (0, 0)),
        pl.BlockSpec((D, N), lambda: (0, 0)),
        pl.BlockSpec((D, D), lambda: (0, 0)),
        pl.BlockSpec((1, D), lambda: (0, 0)),
    ],
    out_specs=pl.BlockSpec((N, D), lambda: (0, 0)),
    out_shape=jax.ShapeDtypeStruct((N, D), f32),
)


def kernel(x, edge_index, W, bias, a, b, c):
    row = edge_index[0].astype(i32)
    col = edge_index[1].astype(i32)
    ep = _tc_pack(row.reshape(EPR, 128), col.reshape(EPR, 128)).reshape(E)
    xT = x.T
    m1f, m2f = _sc_spmm(ep, xT.reshape(D * N))
    cs = jnp.stack([a + b + c, -(2.0 * a + b), a]).astype(f32)
    return _tc_combine(cs, xT, m1f.reshape(D, N), m2f.reshape(D, N),
                       W, bias.reshape(1, D))


# speedup vs baseline: 1.2652x; 1.0398x over previous
import functools

import jax
import jax.numpy as jnp
from jax import lax
from jax.experimental import pallas as pl
from jax.experimental.pallas import tpu as pltpu
from jax.experimental.pallas import tpu_sc as plsc

N = 10000
D = 128
E = 320000

NC = 2
NS = 16
L = 16
NW = NC * NS
FPT = D // NW
NSTRIDE = 10008
NPAD = 10240
NPT = NPAD // NS
CHUNK = 2000
NVEC = CHUNK // L
NCHUNK = E // CHUNK
DEG_E = E // NS
DEG_CH = DEG_E // CHUNK

f32 = jnp.float32
i32 = jnp.int32


def _spl_f(v):
    return jnp.full((L,), v, dtype=f32)


def _spl_i(v):
    return jnp.full((L,), v, dtype=i32)


def _c(v):
    return jnp.int32(v)


def _i(v):
    return v if v.dtype == i32 else lax.convert_element_type(v, i32)


_PERM_DNUMS = lax.GatherDimensionNumbers(
    offset_dims=(), collapsed_slice_dims=(0,), start_index_map=(0,))


def _lperm(v, idx):
    return lax.gather(v, idx.reshape(L, 1), _PERM_DNUMS, (1,),
                      mode=lax.GatherScatterMode.PROMISE_IN_BOUNDS)


def _fori(n, body, unroll=1):
    def outer(j, carry):
        base = j * _c(unroll)
        for u in range(unroll):
            carry = body(base + _c(u), carry)
        return carry
    lax.fori_loop(_c(0), _c(n // unroll), outer, 0)
    for u in range((n // unroll) * unroll, n):
        body(_c(u), 0)


def _sc_body(ep_hbm, xT_hbm, m1T_hbm, m2T_hbm,
             x_sl, agg, deg_priv, dinv, red_buf, ep0, ep1,
             deg_parts, dinv_s, sem0, sem1, sem_x):
    c = _i(lax.axis_index("c"))
    s = _i(lax.axis_index("s"))
    wid = s * _c(NC) + c
    f0 = wid * _c(FPT)
    sems = (sem0, sem1)
    ebufs = (ep0, ep1)
    c14 = _spl_i(14)
    cmask = _spl_i((1 << 14) - 1)

    xcopies = [
        pltpu.make_async_copy(xT_hbm.at[pl.ds((f0 + _c(k)) * _c(N), N)],
                              x_sl.at[pl.ds(_c(k * NSTRIDE), N)], sem_x)
        for k in range(FPT)
    ]
    for cp in xcopies:
        cp.start()

    def zero_deg(j, carry):
        deg_priv[pl.ds(j * _c(L), L)] = _spl_f(0.0)
        return carry

    _fori(NPAD // L, zero_deg, unroll=4)

    def zero_agg(j, carry):
        agg[pl.ds(j * _c(L), L)] = _spl_f(0.0)
        return carry

    _fori(FPT * NSTRIDE // L, zero_agg, unroll=4)

    j16 = lax.iota(i32, 16)
    e8 = lax.shift_right_logical(j16, _spl_i(1))
    phs = (e8, e8 + _spl_i(8))
    kp01 = (j16 & _spl_i(1)) * _spl_i(NSTRIDE)
    kps = (kp01, kp01 + _spl_i(2 * NSTRIDE))

    def edge_sweep(nchunks, base, process_vec):
        def start(chunk_idx, b):
            off = base + chunk_idx * _c(CHUNK)
            pltpu.make_async_copy(ep_hbm.at[pl.ds(off, CHUNK)],
                                  ebufs[b], sems[b]).start()

        def wait(b):
            pltpu.make_async_copy(ep_hbm.at[pl.ds(_c(0), CHUNK)],
                                  ebufs[b], sems[b]).wait()

        def proc_buf(b):
            @plsc.parallel_loop(_c(0), _c(NVEC), _c(1), unroll=5)
            def _(j):
                process_vec(b, j)

        last = _c(nchunks - 1)
        start(_c(0), 0)

        def obody(i, carry):
            g = i * _c(2)
            start(jnp.minimum(g + _c(1), last), 1)
            wait(0)
            proc_buf(0)
            start(jnp.minimum(g + _c(2), last), 0)
            wait(1)
            proc_buf(1)
            return carry

        _fori(nchunks // 2, obody)
        wait(0)

    def deg_vec(b, j):
        sl = pl.ds(j * _c(L), L)
        ep = ebufs[b][sl]
        r = lax.shift_right_logical(ep, c14)
        cl = ep & cmask
        m = jnp.where(r != cl, _spl_f(1.0), _spl_f(0.0))
        plsc.addupdate_scatter(deg_priv, [r], m)

    edge_sweep(DEG_CH, s * _c(DEG_E), deg_vec)

    pltpu.sync_copy(deg_priv, deg_parts.at[s])
    plsc.subcore_barrier()
    nb = s * _c(NPT)
    pltpu.sync_copy(deg_parts.at[:, pl.ds(nb, NPT)], red_buf)

    def red_vec(j, carry):
        sl = pl.ds(j * _c(L), L)
        acc = red_buf[_c(0), sl]
        for t in range(1, NS):
            acc = acc + red_buf[_c(t), sl]
        xi = lax.bitcast_convert_type(acc, i32)
        yi = _spl_i(0x5F3759DF) - lax.shift_right_arithmetic(xi, _spl_i(1))
        y = lax.bitcast_convert_type(yi, f32)
        for _ in range(3):
            y = y * (_spl_f(1.5) - _spl_f(0.5) * acc * y * y)
        y = jnp.where(acc > _spl_f(0.5), y, _spl_f(0.0))
        deg_priv[sl] = y
        return carry

    _fori(NPT // L, red_vec)
    pltpu.sync_copy(deg_priv.at[pl.ds(_c(0), NPT)], dinv_s.at[pl.ds(nb, NPT)])
    plsc.subcore_barrier()
    pltpu.sync_copy(dinv_s, dinv)
    for cp in xcopies:
        cp.wait()

    def scale_vec(j, carry):
        sl = pl.ds(j * _c(L), L)
        d = dinv[sl]
        for k in range(FPT):
            off = pl.ds(j * _c(L) + _c(k * NSTRIDE), L)
            x_sl[off] = x_sl[off] * d
        return carry

    _fori(N // L, scale_vec, unroll=2)

    def pass_vec(b, j):
        sl = pl.ds(j * _c(L), L)
        ep = ebufs[b][sl]
        for ph in phs:
            eg = _lperm(ep, ph)
            rg = lax.shift_right_logical(eg, c14)
            cg = eg & cmask
            mf = jnp.where(rg != cg, _spl_f(1.0), _spl_f(0.0))
            for kp in kps:
                xv = plsc.load_gather(x_sl, [cg + kp])
                plsc.addupdate_scatter(agg, [rg + kp], mf * xv)

    def fin_vec(j, carry):
        sl = pl.ds(j * _c(L), L)
        d = dinv[sl]
        for k in range(FPT):
            off = pl.ds(j * _c(L) + _c(k * NSTRIDE), L)
            x_sl[off] = agg[off] * d
            agg[off] = _spl_f(0.0)
        return carry

    def rescale_vec(j, carry):
        sl = pl.ds(j * _c(L), L)
        d = dinv[sl]
        for k in range(FPT):
            off = pl.ds(j * _c(L) + _c(k * NSTRIDE), L)
            x_sl[off] = x_sl[off] * d
        return carry

    edge_sweep(NCHUNK, 0, pass_vec)
    _fori(N // L, fin_vec, unroll=2)
    for k in range(FPT):
        pltpu.sync_copy(x_sl.at[pl.ds(_c(k * NSTRIDE), N)],
                        m1T_hbm.at[pl.ds((f0 + _c(k)) * _c(N), N)])
    _fori(N // L, rescale_vec, unroll=2)

    edge_sweep(NCHUNK, 0, pass_vec)
    _fori(N // L, fin_vec, unroll=2)
    for k in range(FPT):
        pltpu.sync_copy(x_sl.at[pl.ds(_c(k * NSTRIDE), N)],
                        m2T_hbm.at[pl.ds((f0 + _c(k)) * _c(N), N)])


_sc_spmm = pl.kernel(
    _sc_body,
    out_type=(jax.ShapeDtypeStruct((D * N,), f32),
              jax.ShapeDtypeStruct((D * N,), f32)),
    mesh=plsc.VectorSubcoreMesh(core_axis_name="c", subcore_axis_name="s",
                                num_cores=NC, num_subcores=NS),
    compiler_params=pltpu.CompilerParams(needs_layout_passes=False),
    scratch_types=[
        pltpu.VMEM((FPT * NSTRIDE,), f32),
        pltpu.VMEM((FPT * NSTRIDE,), f32),
        pltpu.VMEM((NPAD,), f32),
        pltpu.VMEM((NPAD,), f32),
        pltpu.VMEM((NS, NPT), f32),
        pltpu.VMEM((CHUNK,), i32),
        pltpu.VMEM((CHUNK,), i32),
        pltpu.VMEM_SHARED((NS, NPAD), f32),
        pltpu.VMEM_SHARED((NPAD,), f32),
        pltpu.SemaphoreType.DMA,
        pltpu.SemaphoreType.DMA,
        pltpu.SemaphoreType.DMA,
    ],
)


BN = 1000


EPR = E // 128


def _pack_body(r_ref, c_ref, out_ref):
    out_ref[...] = jnp.bitwise_or(
        lax.shift_left(r_ref[...], jnp.int32(14)), c_ref[...])


_tc_pack = pl.pallas_call(
    _pack_body,
    in_specs=[pl.BlockSpec((EPR, 128), lambda: (0, 0)),
              pl.BlockSpec((EPR, 128), lambda: (0, 0))],
    out_specs=pl.BlockSpec((EPR, 128), lambda: (0, 0)),
    out_shape=jax.ShapeDtypeStruct((EPR, 128), i32),
)


def _tc_body(cs_ref, xT_ref, m1T_ref, m2T_ref, w_ref, bias_ref, out_ref):
    c0 = cs_ref[0]
    c1 = cs_ref[1]
    c2 = cs_ref[2]
    comb = c0 * xT_ref[...] + c1 * m1T_ref[...] + c2 * m2T_ref[...]
    out_ref[...] = lax.dot_general(
        comb, w_ref[...], (((0,), (1,)), ((), ())),
        preferred_element_type=f32) + bias_ref[...]


_tc_combine = pl.pallas_call(
    _tc_body,
    in_specs=[
        pl.BlockSpec(memory_space=pltpu.SMEM),
        pl.BlockSpec((D, N), lambda: (0, 0)),
        pl.BlockSpec((D, N), lambda: (0, 0)),
        pl.BlockSpec((D, N), lambda: (0, 0)),
        pl.BlockSpec((D, D), lambda: (0, 0)),
        pl.BlockSpec((1, D), lambda: (0, 0)),
    ],
    out_specs=pl.BlockSpec((N, D), lambda: (0, 0)),
    out_shape=jax.ShapeDtypeStruct((N, D), f32),
)


def kernel(x, edge_index, W, bias, a, b, c):
    row = edge_index[0].astype(i32)
    col = edge_index[1].astype(i32)
    ep = _tc_pack(row.reshape(EPR, 128), col.reshape(EPR, 128)).reshape(E)
    xT = x.T
    m1f, m2f = _sc_spmm(ep, xT.reshape(D * N))
    cs = jnp.stack([a + b + c, -(2.0 * a + b), a]).astype(f32)
    return _tc_combine(cs, xT, m1f.reshape(D, N), m2f.reshape(D, N),
                       W, bias.reshape(1, D))


# speedup vs baseline: 1.2799x; 1.0117x over previous
import functools

import jax
import jax.numpy as jnp
from jax import lax
from jax.experimental import pallas as pl
from jax.experimental.pallas import tpu as pltpu
from jax.experimental.pallas import tpu_sc as plsc

N = 10000
D = 128
E = 320000

NC = 2
NS = 16
L = 16
NW = NC * NS
FPT = D // NW
NSTRIDE = 10008
NPAD = 10240
NPT = NPAD // NS
CHUNK = 4000
NVEC = CHUNK // L
NCHUNK = E // CHUNK
DEG_E = E // NS
DEG_CH = DEG_E // CHUNK

f32 = jnp.float32
i32 = jnp.int32


def _spl_f(v):
    return jnp.full((L,), v, dtype=f32)


def _spl_i(v):
    return jnp.full((L,), v, dtype=i32)


def _c(v):
    return jnp.int32(v)


def _i(v):
    return v if v.dtype == i32 else lax.convert_element_type(v, i32)


_PERM_DNUMS = lax.GatherDimensionNumbers(
    offset_dims=(), collapsed_slice_dims=(0,), start_index_map=(0,))


def _lperm(v, idx):
    return lax.gather(v, idx.reshape(L, 1), _PERM_DNUMS, (1,),
                      mode=lax.GatherScatterMode.PROMISE_IN_BOUNDS)


def _fori(n, body, unroll=1):
    def outer(j, carry):
        base = j * _c(unroll)
        for u in range(unroll):
            carry = body(base + _c(u), carry)
        return carry
    lax.fori_loop(_c(0), _c(n // unroll), outer, 0)
    for u in range((n // unroll) * unroll, n):
        body(_c(u), 0)


def _sc_body(ep_hbm, xT_hbm, m1T_hbm, m2T_hbm,
             x_sl, agg, deg_priv, dinv, red_buf, ep0, ep1,
             deg_parts, dinv_s, sem0, sem1, sem_x):
    c = _i(lax.axis_index("c"))
    s = _i(lax.axis_index("s"))
    wid = s * _c(NC) + c
    f0 = wid * _c(FPT)
    sems = (sem0, sem1)
    ebufs = (ep0, ep1)
    c14 = _spl_i(14)
    cmask = _spl_i((1 << 14) - 1)

    xcopies = [
        pltpu.make_async_copy(xT_hbm.at[pl.ds((f0 + _c(k)) * _c(N), N)],
                              x_sl.at[pl.ds(_c(k * NSTRIDE), N)], sem_x)
        for k in range(FPT)
    ]
    for cp in xcopies:
        cp.start()

    def zero_deg(j, carry):
        deg_priv[pl.ds(j * _c(L), L)] = _spl_f(0.0)
        return carry

    _fori(NPAD // L, zero_deg, unroll=4)

    def zero_agg(j, carry):
        agg[pl.ds(j * _c(L), L)] = _spl_f(0.0)
        return carry

    _fori(FPT * NSTRIDE // L, zero_agg, unroll=4)

    j16 = lax.iota(i32, 16)
    e8 = lax.shift_right_logical(j16, _spl_i(1))
    phs = (e8, e8 + _spl_i(8))
    kp01 = (j16 & _spl_i(1)) * _spl_i(NSTRIDE)
    kps = (kp01, kp01 + _spl_i(2 * NSTRIDE))

    def edge_sweep(nchunks, base, process_vec):
        def start(chunk_idx, b):
            off = base + chunk_idx * _c(CHUNK)
            pltpu.make_async_copy(ep_hbm.at[pl.ds(off, CHUNK)],
                                  ebufs[b], sems[b]).start()

        def wait(b):
            pltpu.make_async_copy(ep_hbm.at[pl.ds(_c(0), CHUNK)],
                                  ebufs[b], sems[b]).wait()

        def proc_buf(b):
            @plsc.parallel_loop(_c(0), _c(NVEC), _c(1), unroll=5)
            def _(j):
                process_vec(b, j)

        last = _c(nchunks - 1)
        start(_c(0), 0)

        def obody(i, carry):
            g = i * _c(2)
            start(jnp.minimum(g + _c(1), last), 1)
            wait(0)
            proc_buf(0)
            start(jnp.minimum(g + _c(2), last), 0)
            wait(1)
            proc_buf(1)
            return carry

        _fori(nchunks // 2, obody)
        wait(0)
        if nchunks % 2:
            proc_buf(0)

    def deg_vec(b, j):
        sl = pl.ds(j * _c(L), L)
        ep = ebufs[b][sl]
        r = lax.shift_right_logical(ep, c14)
        cl = ep & cmask
        m = jnp.where(r != cl, _spl_f(1.0), _spl_f(0.0))
        plsc.addupdate_scatter(deg_priv, [r], m)

    edge_sweep(DEG_CH, s * _c(DEG_E), deg_vec)

    pltpu.sync_copy(deg_priv, deg_parts.at[s])
    plsc.subcore_barrier()
    nb = s * _c(NPT)
    pltpu.sync_copy(deg_parts.at[:, pl.ds(nb, NPT)], red_buf)

    def red_vec(j, carry):
        sl = pl.ds(j * _c(L), L)
        acc = red_buf[_c(0), sl]
        for t in range(1, NS):
            acc = acc + red_buf[_c(t), sl]
        xi = lax.bitcast_convert_type(acc, i32)
        yi = _spl_i(0x5F3759DF) - lax.shift_right_arithmetic(xi, _spl_i(1))
        y = lax.bitcast_convert_type(yi, f32)
        for _ in range(3):
            y = y * (_spl_f(1.5) - _spl_f(0.5) * acc * y * y)
        y = jnp.where(acc > _spl_f(0.5), y, _spl_f(0.0))
        deg_priv[sl] = y
        return carry

    _fori(NPT // L, red_vec)
    pltpu.sync_copy(deg_priv.at[pl.ds(_c(0), NPT)], dinv_s.at[pl.ds(nb, NPT)])
    plsc.subcore_barrier()
    pltpu.sync_copy(dinv_s, dinv)
    for cp in xcopies:
        cp.wait()

    def scale_vec(j, carry):
        sl = pl.ds(j * _c(L), L)
        d = dinv[sl]
        for k in range(FPT):
            off = pl.ds(j * _c(L) + _c(k * NSTRIDE), L)
            x_sl[off] = x_sl[off] * d
        return carry

    _fori(N // L, scale_vec, unroll=2)

    def pass_vec(b, j):
        sl = pl.ds(j * _c(L), L)
        ep = ebufs[b][sl]
        for ph in phs:
            eg = _lperm(ep, ph)
            rg = lax.shift_right_logical(eg, c14)
            cg = eg & cmask
            mf = jnp.where(rg != cg, _spl_f(1.0), _spl_f(0.0))
            for kp in kps:
                xv = plsc.load_gather(x_sl, [cg + kp])
                plsc.addupdate_scatter(agg, [rg + kp], mf * xv)

    def fin_vec(j, carry):
        sl = pl.ds(j * _c(L), L)
        d = dinv[sl]
        for k in range(FPT):
            off = pl.ds(j * _c(L) + _c(k * NSTRIDE), L)
            x_sl[off] = agg[off] * d
            agg[off] = _spl_f(0.0)
        return carry

    def rescale_vec(j, carry):
        sl = pl.ds(j * _c(L), L)
        d = dinv[sl]
        for k in range(FPT):
            off = pl.ds(j * _c(L) + _c(k * NSTRIDE), L)
            x_sl[off] = x_sl[off] * d
        return carry

    edge_sweep(NCHUNK, 0, pass_vec)
    _fori(N // L, fin_vec, unroll=2)
    for k in range(FPT):
        pltpu.sync_copy(x_sl.at[pl.ds(_c(k * NSTRIDE), N)],
                        m1T_hbm.at[pl.ds((f0 + _c(k)) * _c(N), N)])
    _fori(N // L, rescale_vec, unroll=2)

    edge_sweep(NCHUNK, 0, pass_vec)
    _fori(N // L, fin_vec, unroll=2)
    for k in range(FPT):
        pltpu.sync_copy(x_sl.at[pl.ds(_c(k * NSTRIDE), N)],
                        m2T_hbm.at[pl.ds((f0 + _c(k)) * _c(N), N)])


_sc_spmm = pl.kernel(
    _sc_body,
    out_type=(jax.ShapeDtypeStruct((D * N,), f32),
              jax.ShapeDtypeStruct((D * N,), f32)),
    mesh=plsc.VectorSubcoreMesh(core_axis_name="c", subcore_axis_name="s",
                                num_cores=NC, num_subcores=NS),
    compiler_params=pltpu.CompilerParams(needs_layout_passes=False),
    scratch_types=[
        pltpu.VMEM((FPT * NSTRIDE,), f32),
        pltpu.VMEM((FPT * NSTRIDE,), f32),
        pltpu.VMEM((NPAD,), f32),
        pltpu.VMEM((NPAD,), f32),
        pltpu.VMEM((NS, NPT), f32),
        pltpu.VMEM((CHUNK,), i32),
        pltpu.VMEM((CHUNK,), i32),
        pltpu.VMEM_SHARED((NS, NPAD), f32),
        pltpu.VMEM_SHARED((NPAD,), f32),
        pltpu.SemaphoreType.DMA,
        pltpu.SemaphoreType.DMA,
        pltpu.SemaphoreType.DMA,
    ],
)


BN = 1000


EPR = E // 128


def _pack_body(r_ref, c_ref, out_ref):
    out_ref[...] = jnp.bitwise_or(
        lax.shift_left(r_ref[...], jnp.int32(14)), c_ref[...])


_tc_pack = pl.pallas_call(
    _pack_body,
    in_specs=[pl.BlockSpec((EPR, 128), lambda: (0, 0)),
              pl.BlockSpec((EPR, 128), lambda: (0, 0))],
    out_specs=pl.BlockSpec((EPR, 128), lambda: (0, 0)),
    out_shape=jax.ShapeDtypeStruct((EPR, 128), i32),
)


def _tc_body(cs_ref, xT_ref, m1T_ref, m2T_ref, w_ref, bias_ref, out_ref):
    c0 = cs_ref[0]
    c1 = cs_ref[1]
    c2 = cs_ref[2]
    comb = c0 * xT_ref[...] + c1 * m1T_ref[...] + c2 * m2T_ref[...]
    out_ref[...] = lax.dot_general(
        comb, w_ref[...], (((0,), (1,)), ((), ())),
        preferred_element_type=f32) + bias_ref[...]


_tc_combine = pl.pallas_call(
    _tc_body,
    in_specs=[
        pl.BlockSpec(memory_space=pltpu.SMEM),
        pl.BlockSpec((D, N), lambda: (0, 0)),
        pl.BlockSpec((D, N), lambda: (0, 0)),
        pl.BlockSpec((D, N), lambda: (0, 0)),
        pl.BlockSpec((D, D), lambda: (0, 0)),
        pl.BlockSpec((1, D), lambda: (0, 0)),
    ],
    out_specs=pl.BlockSpec((N, D), lambda: (0, 0)),
    out_shape=jax.ShapeDtypeStruct((N, D), f32),
)


def kernel(x, edge_index, W, bias, a, b, c):
    row = edge_index[0].astype(i32)
    col = edge_index[1].astype(i32)
    ep = _tc_pack(row.reshape(EPR, 128), col.reshape(EPR, 128)).reshape(E)
    xT = x.T
    m1f, m2f = _sc_spmm(ep, xT.reshape(D * N))
    cs = jnp.stack([a + b + c, -(2.0 * a + b), a]).astype(f32)
    return _tc_combine(cs, xT, m1f.reshape(D, N), m2f.reshape(D, N),
                       W, bias.reshape(1, D))
